# baseline (device time: 20769 ns/iter reference)
import jax
import jax.numpy as jnp
from jax import lax
from jax.experimental import pallas as pl
from jax.experimental.pallas import tpu as pltpu

N_CHUNKS = 8


def kernel(x, pi):
    _, m, n = x.shape
    rows = m // N_CHUNKS

    def body(
        x_hbm,
        pi_ref,
        out_hbm,
        x_stage,
        out_stage,
        send_buf,
        recv_buf,
        scale_send,
        scale_recv,
        load_sems,
        store_sems,
        send_sems,
        recv_sems,
        scale_send_sems,
        scale_recv_sems,
    ):
        my_x = lax.axis_index("x")
        my_y = lax.axis_index("y")
        my_z = lax.axis_index("z")
        partner = (1 - my_x, my_y, my_z)

        barrier = pltpu.get_barrier_semaphore()
        pl.semaphore_signal(
            barrier, inc=1, device_id=partner,
            device_id_type=pl.DeviceIdType.MESH,
        )
        pl.semaphore_wait(barrier, 1)

        swap = pi_ref[my_x] != my_x

        def load_dma(c):
            return pltpu.make_async_copy(
                x_hbm.at[0, pl.ds(c * rows, rows), :],
                x_stage.at[c % 2],
                load_sems.at[c % 2],
            )

        def store_dma(c):
            return pltpu.make_async_copy(
                out_stage.at[c % 2],
                out_hbm.at[0, pl.ds(c * rows, rows), :],
                store_sems.at[c % 2],
            )

        def chunk_rdma(c):
            return pltpu.make_async_remote_copy(
                src_ref=send_buf.at[c],
                dst_ref=recv_buf.at[c],
                send_sem=send_sems.at[c],
                recv_sem=recv_sems.at[c],
                device_id=partner,
                device_id_type=pl.DeviceIdType.MESH,
            )

        def scale_rdma(c):
            return pltpu.make_async_remote_copy(
                src_ref=scale_send.at[c],
                dst_ref=scale_recv.at[c],
                send_sem=scale_send_sems.at[c],
                recv_sem=scale_recv_sems.at[c],
                device_id=partner,
                device_id_type=pl.DeviceIdType.MESH,
            )

        @pl.when(swap)
        def _():
            load_dma(0).start()
            load_dma(1).start()
            for c in range(N_CHUNKS):
                load_dma(c).wait()
                chunk = x_stage[c % 2]
                amax = jnp.max(jnp.abs(chunk))
                scale = jnp.maximum(amax, 1e-30) / 127.0
                scale_send[c] = jnp.full((128,), scale, jnp.float32)
                scale_rdma(c).start()
                q = jnp.round(chunk * (1.0 / scale))
                send_buf[c] = jnp.clip(q, -127.0, 127.0).astype(jnp.int8)
                chunk_rdma(c).start()
                if c + 2 < N_CHUNKS:
                    load_dma(c + 2).start()
            for c in range(N_CHUNKS):
                scale_rdma(c).wait_recv()
                chunk_rdma(c).wait_recv()
                if c >= 2:
                    store_dma(c - 2).wait()
                out_stage[c % 2] = (
                    recv_buf[c].astype(jnp.float32) * scale_recv[c, 0]
                )
                store_dma(c).start()
            store_dma(N_CHUNKS - 2).wait()
            store_dma(N_CHUNKS - 1).wait()
            for c in range(N_CHUNKS):
                scale_rdma(c).wait_send()
                chunk_rdma(c).wait_send()

        @pl.when(jnp.logical_not(swap))
        def _():
            for c in range(N_CHUNKS):
                load_dma(c).start()
                load_dma(c).wait()
                out_stage[c % 2] = x_stage[c % 2]
                store_dma(c).start()
                store_dma(c).wait()

    return pl.pallas_call(
        body,
        out_shape=jax.ShapeDtypeStruct(x.shape, jnp.float32),
        in_specs=[
            pl.BlockSpec(memory_space=pl.ANY),
            pl.BlockSpec(memory_space=pltpu.SMEM),
        ],
        out_specs=pl.BlockSpec(memory_space=pl.ANY),
        scratch_shapes=[
            pltpu.VMEM((2, rows, n), jnp.float32),
            pltpu.VMEM((2, rows, n), jnp.float32),
            pltpu.VMEM((N_CHUNKS, rows, n), jnp.int8),
            pltpu.VMEM((N_CHUNKS, rows, n), jnp.int8),
            pltpu.VMEM((N_CHUNKS, 128), jnp.float32),
            pltpu.VMEM((N_CHUNKS, 128), jnp.float32),
            pltpu.SemaphoreType.DMA((2,)),
            pltpu.SemaphoreType.DMA((2,)),
            pltpu.SemaphoreType.DMA((N_CHUNKS,)),
            pltpu.SemaphoreType.DMA((N_CHUNKS,)),
            pltpu.SemaphoreType.DMA((N_CHUNKS,)),
            pltpu.SemaphoreType.DMA((N_CHUNKS,)),
        ],
        compiler_params=pltpu.CompilerParams(collective_id=0),
    )(x, pi)


# device time: 20615 ns/iter; 1.0075x vs baseline; 1.0075x over previous
import jax
import jax.numpy as jnp
from jax import lax
from jax.experimental import pallas as pl
from jax.experimental.pallas import tpu as pltpu

N_CHUNKS = 8


def kernel(x, pi):
    _, m, n = x.shape
    rows = m // N_CHUNKS

    def body(
        x_ref,
        pi_ref,
        out_ref,
        send_buf,
        recv_buf,
        scale_send,
        scale_recv,
        send_sems,
        recv_sems,
        scale_send_sems,
        scale_recv_sems,
    ):
        my_x = lax.axis_index("x")
        my_y = lax.axis_index("y")
        my_z = lax.axis_index("z")
        partner = (1 - my_x, my_y, my_z)

        barrier = pltpu.get_barrier_semaphore()
        pl.semaphore_signal(
            barrier, inc=1, device_id=partner,
            device_id_type=pl.DeviceIdType.MESH,
        )
        pl.semaphore_wait(barrier, 1)

        swap = pi_ref[my_x] != my_x

        def chunk_rdma(c):
            return pltpu.make_async_remote_copy(
                src_ref=send_buf.at[c],
                dst_ref=recv_buf.at[c],
                send_sem=send_sems.at[c],
                recv_sem=recv_sems.at[c],
                device_id=partner,
                device_id_type=pl.DeviceIdType.MESH,
            )

        def scale_rdma(c):
            return pltpu.make_async_remote_copy(
                src_ref=scale_send.at[c],
                dst_ref=scale_recv.at[c],
                send_sem=scale_send_sems.at[c],
                recv_sem=scale_recv_sems.at[c],
                device_id=partner,
                device_id_type=pl.DeviceIdType.MESH,
            )

        @pl.when(swap)
        def _():
            for c in range(N_CHUNKS):
                chunk = x_ref[0, pl.ds(c * rows, rows), :]
                amax = jnp.max(jnp.abs(chunk))
                scale = jnp.maximum(amax, 1e-30) / 127.0
                scale_send[c] = jnp.full((128,), scale, jnp.float32)
                scale_rdma(c).start()
                q = jnp.round(chunk * (1.0 / scale))
                send_buf[c] = jnp.clip(q, -127.0, 127.0).astype(jnp.int8)
                chunk_rdma(c).start()
            for c in range(N_CHUNKS):
                scale_rdma(c).wait_recv()
                chunk_rdma(c).wait_recv()
                out_ref[0, pl.ds(c * rows, rows), :] = (
                    recv_buf[c].astype(jnp.float32) * scale_recv[c, 0]
                )
            for c in range(N_CHUNKS):
                scale_rdma(c).wait_send()
                chunk_rdma(c).wait_send()

        @pl.when(jnp.logical_not(swap))
        def _():
            out_ref[...] = x_ref[...]

    return pl.pallas_call(
        body,
        out_shape=jax.ShapeDtypeStruct(x.shape, jnp.float32),
        in_specs=[
            pl.BlockSpec(memory_space=pltpu.VMEM),
            pl.BlockSpec(memory_space=pltpu.SMEM),
        ],
        out_specs=pl.BlockSpec(memory_space=pltpu.VMEM),
        scratch_shapes=[
            pltpu.VMEM((N_CHUNKS, rows, n), jnp.int8),
            pltpu.VMEM((N_CHUNKS, rows, n), jnp.int8),
            pltpu.VMEM((N_CHUNKS, 128), jnp.float32),
            pltpu.VMEM((N_CHUNKS, 128), jnp.float32),
            pltpu.SemaphoreType.DMA((N_CHUNKS,)),
            pltpu.SemaphoreType.DMA((N_CHUNKS,)),
            pltpu.SemaphoreType.DMA((N_CHUNKS,)),
            pltpu.SemaphoreType.DMA((N_CHUNKS,)),
        ],
        compiler_params=pltpu.CompilerParams(collective_id=0),
    )(x, pi)
